# Initial kernel scaffold; baseline (speedup 1.0000x reference)
import jax
import jax.numpy as jnp
from jax.experimental import pallas as pl


def kernel(z_bill_version, z_bill, z_legislator_term, z_legislator, src_is_version, dst_is_version, src_same_person, dst_same_person):
    raise NotImplementedError("write your pallas kernel here")


# speedup vs baseline: 4.5921x; 4.5921x over previous
import functools

import jax
import jax.numpy as jnp
from jax import lax
from jax.experimental import pallas as pl
from jax.experimental.pallas import tpu as pltpu
from jax.experimental.pallas import tpu_sc as plsc

N_SRC = 100000
N_DST = 20000
E = 640000
D = 128
NSC = 2
PER_SC = N_DST // NSC
CHUNK = 128
NCHUNKS = E // CHUNK
NTILES = 16
FCHUNK = 80
NFCHUNKS = PER_SC // FCHUNK


def _agg_body(table, zdst, src_idx, dst_idx, out,
              idx_s, idx_d, idxg, idxl, rows, ones_v,
              sbuf, cbuf, obuf, sums_sh, cnt_sh, sem):
    c = lax.axis_index("c")
    s = lax.axis_index("s")
    base_dst = c * PER_SC

    zeros16 = jnp.zeros((16,), jnp.float32)
    for j in range(CHUNK // 16):
        ones_v[pl.ds(j * 16, 16)] = jnp.ones((16,), jnp.float32)

    def zrow_init(i, _):
        for j in range(D // 16):
            obuf[i, pl.ds(j * 16, 16)] = zeros16
        return _
    lax.fori_loop(0, FCHUNK, zrow_init, None)

    def cb_init(i, _):
        cbuf[pl.ds(i * 16, 16)] = zeros16
        return _
    lax.fori_loop(0, FCHUNK // 16, cb_init, None)

    nk = (NFCHUNKS - s + NTILES - 1) // NTILES

    def zero_step(t, _):
        r = pl.multiple_of((s + t * NTILES) * FCHUNK, 8)
        pltpu.sync_copy(obuf, sums_sh.at[pl.ds(r, FCHUNK)])
        pltpu.sync_copy(cbuf, cnt_sh.at[pl.ds(r, FCHUNK)])
        return _
    lax.fori_loop(0, nk, zero_step, None)

    plsc.subcore_barrier()

    lo = (s * NCHUNKS) // NTILES
    hi = ((s + 1) * NCHUNKS) // NTILES

    def edge_step(i, _):
        base = pl.multiple_of(i * CHUNK, CHUNK)
        pltpu.sync_copy(src_idx.at[pl.ds(base, CHUNK)], idx_s)
        pltpu.sync_copy(dst_idx.at[pl.ds(base, CHUNK)], idx_d)
        for j in range(CHUNK // 16):
            sl = pl.ds(j * 16, 16)
            dl = idx_d[sl] - base_dst
            ok = (dl >= 0) & (dl < PER_SC)
            idxl[sl] = jnp.where(ok, dl, -1)
            idxg[sl] = jnp.where(ok, idx_s[sl], -1)
        pltpu.async_copy(
            table.at[plsc.Indices(idxg, ignored_value=-1)], rows, sem).wait()
        pltpu.sync_copy(rows, sums_sh.at[plsc.Indices(idxl, ignored_value=-1)],
                        add=True)
        pltpu.sync_copy(ones_v, cnt_sh.at[plsc.Indices(idxl, ignored_value=-1)],
                        add=True)
        return _

    lax.fori_loop(lo, hi, edge_step, None)

    plsc.subcore_barrier()

    def fin_step(t, _):
        rl = pl.multiple_of((s + t * NTILES) * FCHUNK, 8)
        rg = pl.multiple_of(base_dst + rl, 8)
        pltpu.sync_copy(sums_sh.at[pl.ds(rl, FCHUNK)], sbuf)
        pltpu.sync_copy(cnt_sh.at[pl.ds(rl, FCHUNK)], cbuf)
        pltpu.sync_copy(zdst.at[pl.ds(rg, FCHUNK)], obuf)

        def row_block(b, _):
            i0 = pl.multiple_of(b * 16, 16)
            inv16 = 0.3 / jnp.maximum(cbuf[pl.ds(i0, 16)], 1.0)
            for lane in range(16):
                i = i0 + lane
                inv = inv16[lane]
                for j in range(D // 16):
                    sl = pl.ds(j * 16, 16)
                    obuf[i, sl] = obuf[i, sl] * 0.7 + sbuf[i, sl] * inv
            return _
        lax.fori_loop(0, FCHUNK // 16, row_block, None)
        pltpu.sync_copy(obuf, out.at[pl.ds(rg, FCHUNK)])
        return _

    lax.fori_loop(0, nk, fin_step, None)


@functools.partial(
    pl.kernel,
    out_type=jax.ShapeDtypeStruct((N_DST, D), jnp.float32),
    mesh=plsc.VectorSubcoreMesh(core_axis_name="c", subcore_axis_name="s"),
    scratch_types=[
        pltpu.VMEM((CHUNK,), jnp.int32),
        pltpu.VMEM((CHUNK,), jnp.int32),
        pltpu.VMEM((CHUNK,), jnp.int32),
        pltpu.VMEM((CHUNK,), jnp.int32),
        pltpu.VMEM((CHUNK, D), jnp.float32),
        pltpu.VMEM((CHUNK,), jnp.float32),
        pltpu.VMEM((FCHUNK, D), jnp.float32),
        pltpu.VMEM((FCHUNK,), jnp.float32),
        pltpu.VMEM((FCHUNK, D), jnp.float32),
        pltpu.VMEM_SHARED((PER_SC, D), jnp.float32),
        pltpu.VMEM_SHARED((PER_SC,), jnp.float32),
        pltpu.SemaphoreType.DMA,
    ],
)
def _agg_call(table, zdst, src_idx, dst_idx, out, *scratch):
    _agg_body(table, zdst, src_idx, dst_idx, out, *scratch)


@jax.jit
def _run(z_bill_version, z_bill, z_legislator_term, z_legislator,
         src_is_version, dst_is_version, src_same_person, dst_same_person):
    out_b = _agg_call(z_bill_version, z_bill, src_is_version, dst_is_version)
    out_l = _agg_call(z_legislator_term, z_legislator,
                      src_same_person, dst_same_person)
    return out_b, out_l


def kernel(z_bill_version, z_bill, z_legislator_term, z_legislator,
           src_is_version, dst_is_version, src_same_person, dst_same_person):
    return _run(z_bill_version, z_bill, z_legislator_term, z_legislator,
                src_is_version, dst_is_version, src_same_person,
                dst_same_person)


# speedup vs baseline: 6.0476x; 1.3170x over previous
import functools

import jax
import jax.numpy as jnp
from jax import lax
from jax.experimental import pallas as pl
from jax.experimental.pallas import tpu as pltpu
from jax.experimental.pallas import tpu_sc as plsc

N_SRC = 100000
N_DST = 20000
E = 640000
D = 128
NSC = 2
PER_SC = N_DST // NSC
CHUNK = 256
NCHUNKS = E // CHUNK
NTILES = 16
FCHUNK = 16
NFCHUNKS = PER_SC // FCHUNK


def _agg_body(table, zdst, src_idx, dst_idx, out,
              idx_s, idx_d, idxg, idxl, rows, ones_v,
              sbuf, cbuf, obuf, sums_sh, cnt_sh, sem):
    c = lax.axis_index("c")
    s = lax.axis_index("s")
    base_dst = c * PER_SC

    zeros16 = jnp.zeros((16,), jnp.float32)
    for j in range(CHUNK // 16):
        ones_v[pl.ds(j * 16, 16)] = jnp.ones((16,), jnp.float32)

    def zrow_init(i, _):
        for j in range(D // 16):
            obuf[i, pl.ds(j * 16, 16)] = zeros16
        return _
    lax.fori_loop(0, FCHUNK, zrow_init, None)

    def cb_init(i, _):
        cbuf[pl.ds(i * 16, 16)] = zeros16
        return _
    lax.fori_loop(0, FCHUNK // 16, cb_init, None)

    nk = (NFCHUNKS - s + NTILES - 1) // NTILES

    def zero_step(t, _):
        r = pl.multiple_of((s + t * NTILES) * FCHUNK, 8)
        pltpu.sync_copy(obuf, sums_sh.at[pl.ds(r, FCHUNK)])
        pltpu.sync_copy(cbuf, cnt_sh.at[pl.ds(r, FCHUNK)])
        return _
    lax.fori_loop(0, nk, zero_step, None)

    plsc.subcore_barrier()

    lo = (s * NCHUNKS) // NTILES
    hi = ((s + 1) * NCHUNKS) // NTILES

    def edge_step(i, _):
        base = pl.multiple_of(i * CHUNK, CHUNK)
        pltpu.sync_copy(src_idx.at[pl.ds(base, CHUNK)], idx_s)
        pltpu.sync_copy(dst_idx.at[pl.ds(base, CHUNK)], idx_d)
        for j in range(CHUNK // 16):
            sl = pl.ds(j * 16, 16)
            dl = idx_d[sl] - base_dst
            ok = (dl >= 0) & (dl < PER_SC)
            idxl[sl] = jnp.where(ok, dl, -1)
            idxg[sl] = jnp.where(ok, idx_s[sl], -1)
        pltpu.async_copy(
            table.at[plsc.Indices(idxg, ignored_value=-1)], rows, sem).wait()
        pltpu.sync_copy(rows, sums_sh.at[plsc.Indices(idxl, ignored_value=-1)],
                        add=True)
        pltpu.sync_copy(ones_v, cnt_sh.at[plsc.Indices(idxl, ignored_value=-1)],
                        add=True)
        return _

    lax.fori_loop(lo, hi, edge_step, None)

    plsc.subcore_barrier()

    def fin_step(t, _):
        rl = pl.multiple_of((s + t * NTILES) * FCHUNK, 8)
        rg = pl.multiple_of(base_dst + rl, 8)
        pltpu.sync_copy(sums_sh.at[pl.ds(rl, FCHUNK)], sbuf)
        pltpu.sync_copy(cnt_sh.at[pl.ds(rl, FCHUNK)], cbuf)
        pltpu.sync_copy(zdst.at[pl.ds(rg, FCHUNK)], obuf)

        def row_block(b, _):
            i0 = pl.multiple_of(b * 16, 16)
            inv16 = 0.3 / jnp.maximum(cbuf[pl.ds(i0, 16)], 1.0)
            for lane in range(16):
                i = i0 + lane
                inv = inv16[lane]
                for j in range(D // 16):
                    sl = pl.ds(j * 16, 16)
                    obuf[i, sl] = obuf[i, sl] * 0.7 + sbuf[i, sl] * inv
            return _
        lax.fori_loop(0, FCHUNK // 16, row_block, None)
        pltpu.sync_copy(obuf, out.at[pl.ds(rg, FCHUNK)])
        return _

    lax.fori_loop(0, nk, fin_step, None)


@functools.partial(
    pl.kernel,
    out_type=jax.ShapeDtypeStruct((N_DST, D), jnp.float32),
    mesh=plsc.VectorSubcoreMesh(core_axis_name="c", subcore_axis_name="s"),
    scratch_types=[
        pltpu.VMEM((CHUNK,), jnp.int32),
        pltpu.VMEM((CHUNK,), jnp.int32),
        pltpu.VMEM((CHUNK,), jnp.int32),
        pltpu.VMEM((CHUNK,), jnp.int32),
        pltpu.VMEM((CHUNK, D), jnp.float32),
        pltpu.VMEM((CHUNK,), jnp.float32),
        pltpu.VMEM((FCHUNK, D), jnp.float32),
        pltpu.VMEM((FCHUNK,), jnp.float32),
        pltpu.VMEM((FCHUNK, D), jnp.float32),
        pltpu.VMEM_SHARED((PER_SC, D), jnp.float32),
        pltpu.VMEM_SHARED((PER_SC,), jnp.float32),
        pltpu.SemaphoreType.DMA,
    ],
)
def _agg_call(table, zdst, src_idx, dst_idx, out, *scratch):
    _agg_body(table, zdst, src_idx, dst_idx, out, *scratch)


@jax.jit
def _run(z_bill_version, z_bill, z_legislator_term, z_legislator,
         src_is_version, dst_is_version, src_same_person, dst_same_person):
    out_b = _agg_call(z_bill_version, z_bill, src_is_version, dst_is_version)
    out_l = _agg_call(z_legislator_term, z_legislator,
                      src_same_person, dst_same_person)
    return out_b, out_l


def kernel(z_bill_version, z_bill, z_legislator_term, z_legislator,
           src_is_version, dst_is_version, src_same_person, dst_same_person):
    return _run(z_bill_version, z_bill, z_legislator_term, z_legislator,
                src_is_version, dst_is_version, src_same_person,
                dst_same_person)


# speedup vs baseline: 7.3058x; 1.2081x over previous
import functools

import jax
import jax.numpy as jnp
from jax import lax
from jax.experimental import pallas as pl
from jax.experimental.pallas import tpu as pltpu
from jax.experimental.pallas import tpu_sc as plsc

N_SRC = 100000
N_DST = 20000
E = 640000
D = 128
NSC = 2
PER_SC = N_DST // NSC
CHUNK = 160
NCHUNKS = E // CHUNK
NPAIRS = NCHUNKS // 2
NTILES = 16
FCHUNK = 16
NFCHUNKS = PER_SC // FCHUNK


def _agg_body(table, zdst, src_idx, dst_idx, out,
              idx_s, idx_d, idxg0, idxl0, idxg1, idxl1, rows0, rows1, ones_v,
              sbuf, cbuf, obuf, sums_sh, cnt_sh, gsem0, gsem1):
    c = lax.axis_index("c")
    s = lax.axis_index("s")
    base_dst = c * PER_SC

    zeros16 = jnp.zeros((16,), jnp.float32)
    for j in range(CHUNK // 16):
        ones_v[pl.ds(j * 16, 16)] = jnp.ones((16,), jnp.float32)

    def zrow_init(i, _):
        for j in range(D // 16):
            obuf[i, pl.ds(j * 16, 16)] = zeros16
        return _
    lax.fori_loop(0, FCHUNK, zrow_init, None)

    def cb_init(i, _):
        cbuf[pl.ds(i * 16, 16)] = zeros16
        return _
    lax.fori_loop(0, FCHUNK // 16, cb_init, None)

    nk = (NFCHUNKS - s + NTILES - 1) // NTILES

    def zero_step(t, _):
        r = pl.multiple_of((s + t * NTILES) * FCHUNK, 8)
        pltpu.sync_copy(obuf, sums_sh.at[pl.ds(r, FCHUNK)])
        pltpu.sync_copy(cbuf, cnt_sh.at[pl.ds(r, FCHUNK)])
        return _
    lax.fori_loop(0, nk, zero_step, None)

    plsc.subcore_barrier()

    plo = (s * NPAIRS) // NTILES
    phi = ((s + 1) * NPAIRS) // NTILES

    def stage(i, idxg, idxl):
        base = pl.multiple_of(i * CHUNK, CHUNK)
        pltpu.sync_copy(src_idx.at[pl.ds(base, CHUNK)], idx_s)
        pltpu.sync_copy(dst_idx.at[pl.ds(base, CHUNK)], idx_d)
        for j in range(CHUNK // 16):
            sl = pl.ds(j * 16, 16)
            dl = idx_d[sl] - base_dst
            ok = (dl >= 0) & (dl < PER_SC)
            idxl[sl] = jnp.where(ok, dl, -1)
            idxg[sl] = jnp.where(ok, idx_s[sl], -1)

    def scatter(rows, idxl):
        pltpu.sync_copy(rows, sums_sh.at[plsc.Indices(idxl, ignored_value=-1)],
                        add=True)
        pltpu.sync_copy(ones_v, cnt_sh.at[plsc.Indices(idxl, ignored_value=-1)],
                        add=True)

    def pair_step(t, _):
        i0 = 2 * t
        stage(i0, idxg0, idxl0)
        ga = pltpu.async_copy(
            table.at[plsc.Indices(idxg0, ignored_value=-1)], rows0, gsem0)
        stage(i0 + 1, idxg1, idxl1)
        gb = pltpu.async_copy(
            table.at[plsc.Indices(idxg1, ignored_value=-1)], rows1, gsem1)
        ga.wait()
        scatter(rows0, idxl0)
        gb.wait()
        scatter(rows1, idxl1)
        return _

    lax.fori_loop(plo, phi, pair_step, None)

    plsc.subcore_barrier()

    def fin_step(t, _):
        rl = pl.multiple_of((s + t * NTILES) * FCHUNK, 8)
        rg = pl.multiple_of(base_dst + rl, 8)
        pltpu.sync_copy(sums_sh.at[pl.ds(rl, FCHUNK)], sbuf)
        pltpu.sync_copy(cnt_sh.at[pl.ds(rl, FCHUNK)], cbuf)
        pltpu.sync_copy(zdst.at[pl.ds(rg, FCHUNK)], obuf)

        def row_block(b, _):
            i0 = pl.multiple_of(b * 16, 16)
            inv16 = 0.3 / jnp.maximum(cbuf[pl.ds(i0, 16)], 1.0)
            for lane in range(16):
                i = i0 + lane
                inv = inv16[lane]
                for j in range(D // 16):
                    sl = pl.ds(j * 16, 16)
                    obuf[i, sl] = obuf[i, sl] * 0.7 + sbuf[i, sl] * inv
            return _
        lax.fori_loop(0, FCHUNK // 16, row_block, None)
        pltpu.sync_copy(obuf, out.at[pl.ds(rg, FCHUNK)])
        return _

    lax.fori_loop(0, nk, fin_step, None)


@functools.partial(
    pl.kernel,
    out_type=jax.ShapeDtypeStruct((N_DST, D), jnp.float32),
    mesh=plsc.VectorSubcoreMesh(core_axis_name="c", subcore_axis_name="s"),
    scratch_types=[
        pltpu.VMEM((CHUNK,), jnp.int32),
        pltpu.VMEM((CHUNK,), jnp.int32),
        pltpu.VMEM((CHUNK,), jnp.int32),
        pltpu.VMEM((CHUNK,), jnp.int32),
        pltpu.VMEM((CHUNK,), jnp.int32),
        pltpu.VMEM((CHUNK,), jnp.int32),
        pltpu.VMEM((CHUNK, D), jnp.float32),
        pltpu.VMEM((CHUNK, D), jnp.float32),
        pltpu.VMEM((CHUNK,), jnp.float32),
        pltpu.VMEM((FCHUNK, D), jnp.float32),
        pltpu.VMEM((FCHUNK,), jnp.float32),
        pltpu.VMEM((FCHUNK, D), jnp.float32),
        pltpu.VMEM_SHARED((PER_SC, D), jnp.float32),
        pltpu.VMEM_SHARED((PER_SC,), jnp.float32),
        pltpu.SemaphoreType.DMA,
        pltpu.SemaphoreType.DMA,
    ],
)
def _agg_call(table, zdst, src_idx, dst_idx, out, *scratch):
    _agg_body(table, zdst, src_idx, dst_idx, out, *scratch)


@jax.jit
def _run(z_bill_version, z_bill, z_legislator_term, z_legislator,
         src_is_version, dst_is_version, src_same_person, dst_same_person):
    out_b = _agg_call(z_bill_version, z_bill, src_is_version, dst_is_version)
    out_l = _agg_call(z_legislator_term, z_legislator,
                      src_same_person, dst_same_person)
    return out_b, out_l


def kernel(z_bill_version, z_bill, z_legislator_term, z_legislator,
           src_is_version, dst_is_version, src_same_person, dst_same_person):
    return _run(z_bill_version, z_bill, z_legislator_term, z_legislator,
                src_is_version, dst_is_version, src_same_person,
                dst_same_person)


# speedup vs baseline: 8.3202x; 1.1388x over previous
import functools

import jax
import jax.numpy as jnp
from jax import lax
from jax.experimental import pallas as pl
from jax.experimental.pallas import tpu as pltpu
from jax.experimental.pallas import tpu_sc as plsc

N_SRC = 100000
N_DST = 20000
E = 640000
D = 128
NSC = 2
PER_SC = N_DST // NSC
CHUNK = 160
NCHUNKS = E // CHUNK
NPAIRS = NCHUNKS // 2
NTILES = 16
FCHUNK = 16
NFCHUNKS = PER_SC // FCHUNK


def _agg_body(table, zdst, src_idx, dst_idx, out,
              idx_s, idx_d, idxg0, idxl0, idxg1, idxl1, rows0, rows1, ones_v,
              sbuf, cbuf, obuf, sums_sh, cnt_sh, gsem0, gsem1, ssem0, ssem1):
    c = lax.axis_index("c")
    s = lax.axis_index("s")
    base_dst = c * PER_SC

    zeros16 = jnp.zeros((16,), jnp.float32)
    for j in range(CHUNK // 16):
        ones_v[pl.ds(j * 16, 16)] = jnp.ones((16,), jnp.float32)

    def zrow_init(i, _):
        for j in range(D // 16):
            obuf[i, pl.ds(j * 16, 16)] = zeros16
        return _
    lax.fori_loop(0, FCHUNK, zrow_init, None)

    def cb_init(i, _):
        cbuf[pl.ds(i * 16, 16)] = zeros16
        return _
    lax.fori_loop(0, FCHUNK // 16, cb_init, None)

    nk = (NFCHUNKS - s + NTILES - 1) // NTILES

    def zero_step(t, _):
        r = pl.multiple_of((s + t * NTILES) * FCHUNK, 8)
        pltpu.sync_copy(obuf, sums_sh.at[pl.ds(r, FCHUNK)])
        pltpu.sync_copy(cbuf, cnt_sh.at[pl.ds(r, FCHUNK)])
        return _
    lax.fori_loop(0, nk, zero_step, None)

    plsc.subcore_barrier()

    plo = (s * NPAIRS) // NTILES
    phi = ((s + 1) * NPAIRS) // NTILES

    def stage(i, idxg, idxl):
        base = pl.multiple_of(i * CHUNK, CHUNK)
        pltpu.sync_copy(src_idx.at[pl.ds(base, CHUNK)], idx_s)
        pltpu.sync_copy(dst_idx.at[pl.ds(base, CHUNK)], idx_d)
        for j in range(CHUNK // 16):
            sl = pl.ds(j * 16, 16)
            dl = idx_d[sl] - base_dst
            ok = (dl >= 0) & (dl < PER_SC)
            idxl[sl] = jnp.where(ok, dl, -1)
            idxg[sl] = jnp.where(ok, idx_s[sl], -1)

    def scatter(rows, idxl):
        pltpu.sync_copy(rows, sums_sh.at[plsc.Indices(idxl, ignored_value=-1)],
                        add=True)
        pltpu.sync_copy(ones_v, cnt_sh.at[plsc.Indices(idxl, ignored_value=-1)],
                        add=True)

    def gather_start(idxg, rows, gsem):
        pltpu.async_copy(
            table.at[plsc.Indices(idxg, ignored_value=-1)], rows, gsem)

    def gather_wait(idxg, rows, gsem):
        pltpu.make_async_copy(
            table.at[plsc.Indices(idxg, ignored_value=-1)], rows, gsem).wait()

    def scatter_start(rows, idxl, ssem):
        pltpu.async_copy(
            rows, sums_sh.at[plsc.Indices(idxl, ignored_value=-1)], ssem,
            add=True)
        pltpu.async_copy(
            ones_v, cnt_sh.at[plsc.Indices(idxl, ignored_value=-1)], ssem,
            add=True)

    def scatter_wait(rows, idxl, ssem):
        pltpu.make_async_copy(
            rows, sums_sh.at[plsc.Indices(idxl, ignored_value=-1)],
            ssem).wait()
        pltpu.make_async_copy(
            ones_v, cnt_sh.at[plsc.Indices(idxl, ignored_value=-1)],
            ssem).wait()

    stage(2 * plo, idxg0, idxl0)
    gather_start(idxg0, rows0, gsem0)

    def pair_step(t, _):
        i0 = 2 * t

        @pl.when(t > plo)
        def _():
            scatter_wait(rows1, idxl1, ssem1)

        stage(i0 + 1, idxg1, idxl1)
        gather_wait(idxg0, rows0, gsem0)
        gather_start(idxg1, rows1, gsem1)
        scatter_start(rows0, idxl0, ssem0)
        scatter_wait(rows0, idxl0, ssem0)

        @pl.when(t + 1 < phi)
        def _():
            stage(i0 + 2, idxg0, idxl0)
            gather_start(idxg0, rows0, gsem0)
        gather_wait(idxg1, rows1, gsem1)
        scatter_start(rows1, idxl1, ssem1)
        return _

    lax.fori_loop(plo, phi, pair_step, None)
    scatter_wait(rows1, idxl1, ssem1)

    plsc.subcore_barrier()

    def fin_step(t, _):
        rl = pl.multiple_of((s + t * NTILES) * FCHUNK, 8)
        rg = pl.multiple_of(base_dst + rl, 8)
        pltpu.sync_copy(sums_sh.at[pl.ds(rl, FCHUNK)], sbuf)
        pltpu.sync_copy(cnt_sh.at[pl.ds(rl, FCHUNK)], cbuf)
        pltpu.sync_copy(zdst.at[pl.ds(rg, FCHUNK)], obuf)

        def row_block(b, _):
            i0 = pl.multiple_of(b * 16, 16)
            inv16 = 0.3 / jnp.maximum(cbuf[pl.ds(i0, 16)], 1.0)
            for lane in range(16):
                i = i0 + lane
                inv = inv16[lane]
                for j in range(D // 16):
                    sl = pl.ds(j * 16, 16)
                    obuf[i, sl] = obuf[i, sl] * 0.7 + sbuf[i, sl] * inv
            return _
        lax.fori_loop(0, FCHUNK // 16, row_block, None)
        pltpu.sync_copy(obuf, out.at[pl.ds(rg, FCHUNK)])
        return _

    lax.fori_loop(0, nk, fin_step, None)


@functools.partial(
    pl.kernel,
    out_type=jax.ShapeDtypeStruct((N_DST, D), jnp.float32),
    mesh=plsc.VectorSubcoreMesh(core_axis_name="c", subcore_axis_name="s"),
    scratch_types=[
        pltpu.VMEM((CHUNK,), jnp.int32),
        pltpu.VMEM((CHUNK,), jnp.int32),
        pltpu.VMEM((CHUNK,), jnp.int32),
        pltpu.VMEM((CHUNK,), jnp.int32),
        pltpu.VMEM((CHUNK,), jnp.int32),
        pltpu.VMEM((CHUNK,), jnp.int32),
        pltpu.VMEM((CHUNK, D), jnp.float32),
        pltpu.VMEM((CHUNK, D), jnp.float32),
        pltpu.VMEM((CHUNK,), jnp.float32),
        pltpu.VMEM((FCHUNK, D), jnp.float32),
        pltpu.VMEM((FCHUNK,), jnp.float32),
        pltpu.VMEM((FCHUNK, D), jnp.float32),
        pltpu.VMEM_SHARED((PER_SC, D), jnp.float32),
        pltpu.VMEM_SHARED((PER_SC,), jnp.float32),
        pltpu.SemaphoreType.DMA,
        pltpu.SemaphoreType.DMA,
        pltpu.SemaphoreType.DMA,
        pltpu.SemaphoreType.DMA,
    ],
)
def _agg_call(table, zdst, src_idx, dst_idx, out, *scratch):
    _agg_body(table, zdst, src_idx, dst_idx, out, *scratch)


@jax.jit
def _run(z_bill_version, z_bill, z_legislator_term, z_legislator,
         src_is_version, dst_is_version, src_same_person, dst_same_person):
    out_b = _agg_call(z_bill_version, z_bill, src_is_version, dst_is_version)
    out_l = _agg_call(z_legislator_term, z_legislator,
                      src_same_person, dst_same_person)
    return out_b, out_l


def kernel(z_bill_version, z_bill, z_legislator_term, z_legislator,
           src_is_version, dst_is_version, src_same_person, dst_same_person):
    return _run(z_bill_version, z_bill, z_legislator_term, z_legislator,
                src_is_version, dst_is_version, src_same_person,
                dst_same_person)
